# baseline (speedup 1.0000x reference)
import functools

import jax
import jax.numpy as jnp
from jax import lax
from jax.experimental import pallas as pl
from jax.experimental.pallas import tpu as pltpu
from jax.experimental.pallas import tpu_sc as plsc

_DIM = 256
_BM = 1024
_BN = 1024
_BM3 = 2048
_W1 = 2736
_W2 = 5472


def _argmin_body(x_ref, xsq_ref, w_ref, idx_ref,
                 rm0, rm1, rm2, ri0, ri1, ri2, xwb, wsqb):
    rm = (rm0, rm1, rm2)
    ri = (ri0, ri1, ri2)
    j = pl.program_id(1)
    nj = pl.num_programs(1) - 1

    @pl.when(j < nj)
    def _():
        x = x_ref[...]
        w = w_ref[...]
        xwb[j % 2] = lax.dot_general(
            x.astype(jnp.bfloat16), w.astype(jnp.bfloat16),
            (((1,), (1,)), ((), ())), preferred_element_type=jnp.float32)
        wsqb[j % 2, 0] = jnp.sum(w * w, axis=1)

    @pl.when(j > 0)
    def _():
        t = j - 1
        bn = xwb.shape[2]
        xw = xwb[t % 2]
        w_sq = wsqb[t % 2, 0]
        d2 = xsq_ref[...] - 2.0 * xw + w_sq[None, :]
        dist = jnp.sqrt(jnp.maximum(d2, 0.0))
        gcol = lax.broadcasted_iota(jnp.int32, dist.shape, 1) + t * bn
        inf = jnp.float32(jnp.inf)

        def minarg(d, cols):
            m = jnp.min(d, axis=1, keepdims=True)
            l = jnp.min(jnp.where(d == m, cols, 8192), axis=1, keepdims=True)
            return m, l

        def init(k, m, l):
            rm[k][...] = m
            ri[k][...] = l

        def update(k, m, l):
            upd = m < rm[k][...]
            ri[k][...] = jnp.where(upd, l, ri[k][...])
            rm[k][...] = jnp.where(upd, m, rm[k][...])

        @pl.when(t == 0)
        def _():
            m, l = minarg(dist, gcol)
            init(0, m, l)

        @pl.when(t == 1)
        def _():
            m, l = minarg(dist, gcol)
            update(0, m, l)

        @pl.when(t == 2)
        def _():
            lo = jnp.where(gcol < _W1, dist, inf)
            m, l = minarg(lo, gcol)
            update(0, m, l)
            hi = jnp.where(gcol >= _W1, dist, inf)
            m, l = minarg(hi, gcol)
            init(1, m, l)

        @pl.when((t == 3) | (t == 4))
        def _():
            m, l = minarg(dist, gcol)
            update(1, m, l)

        @pl.when(t == 5)
        def _():
            lo = jnp.where(gcol < _W2, dist, inf)
            m, l = minarg(lo, gcol)
            update(1, m, l)
            hi = jnp.where(gcol >= _W2, dist, inf)
            m, l = minarg(hi, gcol)
            init(2, m, l)

        @pl.when((t == 6) | (t == 7))
        def _():
            m, l = minarg(dist, gcol)
            update(2, m, l)

        @pl.when(t == nj - 1)
        def _():
            accv = rm[0][...].astype(jnp.bfloat16).astype(jnp.float32)
            acci = ri[0][...]
            b1 = rm[1][...] < accv
            accv = jnp.where(b1, rm[1][...], accv)
            acci = jnp.where(b1, ri[1][...], acci)
            accv = accv.astype(jnp.bfloat16).astype(jnp.float32)
            b2 = rm[2][...] < accv
            acci = jnp.where(b2, ri[2][...], acci)
            idx_ref[...] = acci


def _argmin_call(x, xsq, weight):
    n, d = x.shape
    k = weight.shape[0]
    nj = k // _BN
    grid = (n // _BM, nj + 1)
    return pl.pallas_call(
        _argmin_body,
        grid=grid,
        in_specs=[
            pl.BlockSpec((_BM, d), lambda i, j: (i, 0)),
            pl.BlockSpec((_BM, 1), lambda i, j: (i, 0)),
            pl.BlockSpec((_BN, d), lambda i, j: (jnp.minimum(j, nj - 1), 0)),
        ],
        out_specs=pl.BlockSpec((_BM, 1), lambda i, j: (i, 0)),
        out_shape=jax.ShapeDtypeStruct((n, 1), jnp.int32),
        scratch_shapes=[pltpu.VMEM((_BM, 1), jnp.float32)] * 3
                      + [pltpu.VMEM((_BM, 1), jnp.int32)] * 3
                      + [pltpu.VMEM((2, _BM, _BN), jnp.float32),
                         pltpu.VMEM((2, 1, _BN), jnp.float32)],
        compiler_params=pltpu.CompilerParams(
            dimension_semantics=("parallel", "arbitrary")),
    )(x, xsq, weight)


def _gather_call(weight, idx):
    n = idx.shape[0]
    d = weight.shape[1]
    info = plsc.get_sparse_core_info()
    nw = info.num_cores * info.num_subcores
    b_per_w = n // nw
    ch = min(256, b_per_w)
    mesh = plsc.VectorSubcoreMesh(core_axis_name="c", subcore_axis_name="s")

    @functools.partial(
        pl.kernel, mesh=mesh,
        out_type=jax.ShapeDtypeStruct((n, d), jnp.float32),
        scratch_types=[
            pltpu.VMEM((ch,), jnp.int32),
            pltpu.VMEM((ch, d), jnp.float32),
            pltpu.SemaphoreType.DMA,
        ],
    )
    def gk(table_hbm, idx_hbm, out_hbm, idx_v, rows_v, sem):
        wid = lax.axis_index("s") * info.num_cores + lax.axis_index("c")
        base = wid * b_per_w
        for c in range(b_per_w // ch):
            o = base + c * ch
            pltpu.sync_copy(idx_hbm.at[pl.ds(o, ch)], idx_v)
            pltpu.async_copy(table_hbm.at[idx_v], rows_v, sem).wait()
            pltpu.sync_copy(rows_v, out_hbm.at[pl.ds(o, ch)])

    return gk(weight, idx)


def _st_loss_body(x_ref, q_ref, o_ref, loss_ref):
    i = pl.program_id(0)
    x = x_ref[...]
    q = q_ref[...]
    diff = q - x
    o_ref[...] = x + diff

    @pl.when(i == 0)
    def _():
        loss_ref[...] = jnp.zeros_like(loss_ref)

    loss_ref[...] = loss_ref[...] + jnp.sum(diff * diff)

    @pl.when(i == pl.num_programs(0) - 1)
    def _():
        n_total = x_ref.shape[0] * x_ref.shape[1] * pl.num_programs(0)
        loss_ref[...] = 1.25 * (loss_ref[...] / float(n_total))


def _st_loss_call(x, q):
    n, d = x.shape
    grid = (n // _BM3,)
    return pl.pallas_call(
        _st_loss_body,
        grid=grid,
        in_specs=[
            pl.BlockSpec((_BM3, d), lambda i: (i, 0)),
            pl.BlockSpec((_BM3, d), lambda i: (i, 0)),
        ],
        out_specs=[
            pl.BlockSpec((_BM3, d), lambda i: (i, 0)),
            pl.BlockSpec((1, 1), lambda i: (0, 0)),
        ],
        out_shape=[
            jax.ShapeDtypeStruct((n, d), jnp.float32),
            jax.ShapeDtypeStruct((1, 1), jnp.float32),
        ],
        compiler_params=pltpu.CompilerParams(
            dimension_semantics=("arbitrary",)),
    )(x, q)


def kernel(inputs, weight):
    shp = inputs.shape
    x = inputs.reshape(-1, shp[-1])
    xsq = jnp.sum(x ** 2, axis=1, keepdims=True)
    idx = _argmin_call(x, xsq, weight)
    q = _gather_call(weight, idx.reshape(-1))
    q_st, loss = _st_loss_call(x, q)
    return q_st.reshape(shp), loss[0, 0]


# speedup vs baseline: 1.2597x; 1.2597x over previous
import functools

import jax
import jax.numpy as jnp
from jax import lax
from jax.experimental import pallas as pl
from jax.experimental.pallas import tpu as pltpu
from jax.experimental.pallas import tpu_sc as plsc

_DIM = 256
_BM = 2048
_BN = 1024
_BM3 = 2048
_W1 = 2736
_W2 = 5472


def _argmin_body(x_ref, xsq_ref, w_ref, idx_ref, rm0, rm1, rm2, ri0, ri1, ri2):
    rm = (rm0, rm1, rm2)
    ri = (ri0, ri1, ri2)
    j = pl.program_id(1)
    nj = pl.num_programs(1)
    x = x_ref[...]
    w = w_ref[...]
    bn = w.shape[0]
    xw = lax.dot_general(x.astype(jnp.bfloat16), w.astype(jnp.bfloat16),
                         (((1,), (1,)), ((), ())),
                         preferred_element_type=jnp.float32)
    w_sq = jnp.sum(w * w, axis=1)
    d2 = xsq_ref[...] - 2.0 * xw + w_sq[None, :]
    dist = jnp.sqrt(jnp.maximum(d2, 0.0))
    gcol = lax.broadcasted_iota(jnp.int32, dist.shape, 1) + j * bn
    inf = jnp.float32(jnp.inf)

    def minarg(d, cols):
        m = jnp.min(d, axis=1, keepdims=True)
        l = jnp.min(jnp.where(d == m, cols, 8192), axis=1, keepdims=True)
        return m, l

    def init(k, m, l):
        rm[k][...] = m
        ri[k][...] = l

    def update(k, m, l):
        upd = m < rm[k][...]
        ri[k][...] = jnp.where(upd, l, ri[k][...])
        rm[k][...] = jnp.where(upd, m, rm[k][...])

    @pl.when(j == 0)
    def _():
        m, l = minarg(dist, gcol)
        init(0, m, l)

    @pl.when(j == 1)
    def _():
        m, l = minarg(dist, gcol)
        update(0, m, l)

    @pl.when(j == 2)
    def _():
        lo = jnp.where(gcol < _W1, dist, inf)
        m, l = minarg(lo, gcol)
        update(0, m, l)
        hi = jnp.where(gcol >= _W1, dist, inf)
        m, l = minarg(hi, gcol)
        init(1, m, l)

    @pl.when((j == 3) | (j == 4))
    def _():
        m, l = minarg(dist, gcol)
        update(1, m, l)

    @pl.when(j == 5)
    def _():
        lo = jnp.where(gcol < _W2, dist, inf)
        m, l = minarg(lo, gcol)
        update(1, m, l)
        hi = jnp.where(gcol >= _W2, dist, inf)
        m, l = minarg(hi, gcol)
        init(2, m, l)

    @pl.when((j == 6) | (j == 7))
    def _():
        m, l = minarg(dist, gcol)
        update(2, m, l)

    @pl.when(j == nj - 1)
    def _():
        accv = rm[0][...].astype(jnp.bfloat16).astype(jnp.float32)
        acci = ri[0][...]
        b1 = rm[1][...] < accv
        accv = jnp.where(b1, rm[1][...], accv)
        acci = jnp.where(b1, ri[1][...], acci)
        accv = accv.astype(jnp.bfloat16).astype(jnp.float32)
        b2 = rm[2][...] < accv
        acci = jnp.where(b2, ri[2][...], acci)
        idx_ref[...] = acci


def _argmin_call(x, xsq, weight):
    n, d = x.shape
    k = weight.shape[0]
    grid = (n // _BM, k // _BN)
    return pl.pallas_call(
        _argmin_body,
        grid=grid,
        in_specs=[
            pl.BlockSpec((_BM, d), lambda i, j: (i, 0)),
            pl.BlockSpec((_BM, 1), lambda i, j: (i, 0)),
            pl.BlockSpec((_BN, d), lambda i, j: (j, 0)),
        ],
        out_specs=pl.BlockSpec((_BM, 1), lambda i, j: (i, 0)),
        out_shape=jax.ShapeDtypeStruct((n, 1), jnp.int32),
        scratch_shapes=[pltpu.VMEM((_BM, 1), jnp.float32)] * 3
                      + [pltpu.VMEM((_BM, 1), jnp.int32)] * 3,
        compiler_params=pltpu.CompilerParams(
            dimension_semantics=("parallel", "arbitrary")),
    )(x, xsq, weight)


def _gather_call(weight, idx):
    n = idx.shape[0]
    d = weight.shape[1]
    info = plsc.get_sparse_core_info()
    nw = info.num_cores * info.num_subcores
    b_per_w = n // nw
    ch = min(256, b_per_w)
    mesh = plsc.VectorSubcoreMesh(core_axis_name="c", subcore_axis_name="s")

    @functools.partial(
        pl.kernel, mesh=mesh,
        out_type=jax.ShapeDtypeStruct((n, d), jnp.float32),
        scratch_types=[
            pltpu.VMEM((ch,), jnp.int32),
            pltpu.VMEM((ch, d), jnp.float32),
            pltpu.SemaphoreType.DMA,
        ],
    )
    def gk(table_hbm, idx_hbm, out_hbm, idx_v, rows_v, sem):
        wid = lax.axis_index("s") * info.num_cores + lax.axis_index("c")
        base = wid * b_per_w
        for c in range(b_per_w // ch):
            o = base + c * ch
            pltpu.sync_copy(idx_hbm.at[pl.ds(o, ch)], idx_v)
            pltpu.async_copy(table_hbm.at[idx_v], rows_v, sem).wait()
            pltpu.sync_copy(rows_v, out_hbm.at[pl.ds(o, ch)])

    return gk(weight, idx)


def _st_loss_body(x_ref, q_ref, o_ref, loss_ref):
    i = pl.program_id(0)
    x = x_ref[...]
    q = q_ref[...]
    diff = q - x
    o_ref[...] = x + diff

    @pl.when(i == 0)
    def _():
        loss_ref[...] = jnp.zeros_like(loss_ref)

    loss_ref[...] = loss_ref[...] + jnp.sum(diff * diff)

    @pl.when(i == pl.num_programs(0) - 1)
    def _():
        n_total = x_ref.shape[0] * x_ref.shape[1] * pl.num_programs(0)
        loss_ref[...] = 1.25 * (loss_ref[...] / float(n_total))


def _st_loss_call(x, q):
    n, d = x.shape
    grid = (n // _BM3,)
    return pl.pallas_call(
        _st_loss_body,
        grid=grid,
        in_specs=[
            pl.BlockSpec((_BM3, d), lambda i: (i, 0)),
            pl.BlockSpec((_BM3, d), lambda i: (i, 0)),
        ],
        out_specs=[
            pl.BlockSpec((_BM3, d), lambda i: (i, 0)),
            pl.BlockSpec((1, 1), lambda i: (0, 0)),
        ],
        out_shape=[
            jax.ShapeDtypeStruct((n, d), jnp.float32),
            jax.ShapeDtypeStruct((1, 1), jnp.float32),
        ],
        compiler_params=pltpu.CompilerParams(
            dimension_semantics=("arbitrary",)),
    )(x, q)


def kernel(inputs, weight):
    shp = inputs.shape
    x = inputs.reshape(-1, shp[-1])
    xsq = jnp.sum(x ** 2, axis=1, keepdims=True)
    idx = _argmin_call(x, xsq, weight)
    q = _gather_call(weight, idx.reshape(-1))
    q_st, loss = _st_loss_call(x, q)
    return q_st.reshape(shp), loss[0, 0]
